# baseline (speedup 1.0000x reference)
import functools

import jax
import jax.numpy as jnp
from jax.experimental import pallas as pl
from jax.experimental.pallas import tpu as pltpu


def _plm_block(itemid_ref, pos_emb_ref, memb_ref,
               pos_out_ref, labels_ref, masked_ref, target_ref, perm_ref,
               *, seq_len: int):
    item = itemid_ref[...]
    bb = item.shape[0]
    hid = pos_emb_ref.shape[2]

    nonpad = (item != 0).astype(jnp.int32)
    last = jnp.sum(nonpad, axis=1, keepdims=True) - 1
    last = jnp.where(last < 0, last + seq_len, last)

    col = jax.lax.broadcasted_iota(jnp.int32, (bb, seq_len), 1)
    is_last = col == last

    labels = jnp.where(is_last, item, 0)
    masked = labels != 0
    labels_ref[...] = labels
    masked_ref[...] = masked

    last_item = jnp.sum(labels, axis=1, keepdims=True)
    col3 = jax.lax.broadcasted_iota(jnp.int32, (bb, seq_len, hid), 1)
    mask3 = (col3 == last[:, :, None]) & (last_item != 0)[:, :, None]

    memb = memb_ref[...]
    pos_out_ref[...] = jnp.where(mask3, memb[None, :, :], pos_emb_ref[...])

    i2 = jax.lax.broadcasted_iota(jnp.int32, (seq_len, seq_len), 0)
    j2 = jax.lax.broadcasted_iota(jnp.int32, (seq_len, seq_len), 1)
    eye = (i2 == j2).astype(jnp.float32)
    target_ref[...] = jnp.broadcast_to(eye[None], (bb, seq_len, seq_len))

    upper = j2 > i2
    perm = upper[None, :, :] | is_last[:, None, :]
    perm_ref[...] = perm.astype(jnp.int32)


def kernel(pos_emb, itemid_seq, training, masked_item_embedding):
    B, S, H = pos_emb.shape
    bb = 8
    memb = masked_item_embedding.reshape(1, H).astype(pos_emb.dtype)

    out = pl.pallas_call(
        functools.partial(_plm_block, seq_len=S),
        grid=(B // bb,),
        in_specs=[
            pl.BlockSpec((bb, S), lambda i: (i, 0)),
            pl.BlockSpec((bb, S, H), lambda i: (i, 0, 0)),
            pl.BlockSpec((1, H), lambda i: (0, 0)),
        ],
        out_specs=[
            pl.BlockSpec((bb, S, H), lambda i: (i, 0, 0)),
            pl.BlockSpec((bb, S), lambda i: (i, 0)),
            pl.BlockSpec((bb, S), lambda i: (i, 0)),
            pl.BlockSpec((bb, S, S), lambda i: (i, 0, 0)),
            pl.BlockSpec((bb, S, S), lambda i: (i, 0, 0)),
        ],
        out_shape=[
            jax.ShapeDtypeStruct((B, S, H), pos_emb.dtype),
            jax.ShapeDtypeStruct((B, S), itemid_seq.dtype),
            jax.ShapeDtypeStruct((B, S), jnp.bool_),
            jax.ShapeDtypeStruct((B, S, S), jnp.float32),
            jax.ShapeDtypeStruct((B, S, S), jnp.int32),
        ],
        compiler_params=pltpu.CompilerParams(
            dimension_semantics=("parallel",),
        ),
    )(itemid_seq, pos_emb, memb)

    pos_emb_inp, labels, masked_labels, target_mapping, perm_mask_out = out
    return (pos_emb_inp, labels, masked_labels, target_mapping, perm_mask_out)


# speedup vs baseline: 1.0314x; 1.0314x over previous
import functools

import jax
import jax.numpy as jnp
from jax.experimental import pallas as pl
from jax.experimental.pallas import tpu as pltpu


def _plm_block(itemid_ref, pos_emb_ref, memb_ref,
               pos_out_ref, labels_ref, masked_ref, target_ref, perm_ref,
               *, seq_len: int):
    item = itemid_ref[...]
    bb = item.shape[0]
    hid = pos_emb_ref.shape[2]

    nonpad = (item != 0).astype(jnp.int32)
    last = jnp.sum(nonpad, axis=1, keepdims=True) - 1
    last = jnp.where(last < 0, last + seq_len, last)

    col = jax.lax.broadcasted_iota(jnp.int32, (bb, seq_len), 1)
    is_last = col == last

    labels = jnp.where(is_last, item, 0)
    masked = labels != 0
    labels_ref[...] = labels
    masked_ref[...] = masked

    last_item = jnp.sum(labels, axis=1, keepdims=True)
    col3 = jax.lax.broadcasted_iota(jnp.int32, (bb, seq_len, hid), 1)
    mask3 = (col3 == last[:, :, None]) & (last_item != 0)[:, :, None]

    memb = memb_ref[...]
    pos_out_ref[...] = jnp.where(mask3, memb[None, :, :], pos_emb_ref[...])

    i2 = jax.lax.broadcasted_iota(jnp.int32, (seq_len, seq_len), 0)
    j2 = jax.lax.broadcasted_iota(jnp.int32, (seq_len, seq_len), 1)
    eye = (i2 == j2).astype(jnp.float32)
    target_ref[...] = jnp.broadcast_to(eye[None], (bb, seq_len, seq_len))

    upper = j2 > i2
    perm = upper[None, :, :] | is_last[:, None, :]
    perm_ref[...] = perm.astype(jnp.int32)


def kernel(pos_emb, itemid_seq, training, masked_item_embedding):
    B, S, H = pos_emb.shape
    bb = 32
    memb = masked_item_embedding.reshape(1, H).astype(pos_emb.dtype)

    out = pl.pallas_call(
        functools.partial(_plm_block, seq_len=S),
        grid=(B // bb,),
        in_specs=[
            pl.BlockSpec((bb, S), lambda i: (i, 0)),
            pl.BlockSpec((bb, S, H), lambda i: (i, 0, 0)),
            pl.BlockSpec((1, H), lambda i: (0, 0)),
        ],
        out_specs=[
            pl.BlockSpec((bb, S, H), lambda i: (i, 0, 0)),
            pl.BlockSpec((bb, S), lambda i: (i, 0)),
            pl.BlockSpec((bb, S), lambda i: (i, 0)),
            pl.BlockSpec((bb, S, S), lambda i: (i, 0, 0)),
            pl.BlockSpec((bb, S, S), lambda i: (i, 0, 0)),
        ],
        out_shape=[
            jax.ShapeDtypeStruct((B, S, H), pos_emb.dtype),
            jax.ShapeDtypeStruct((B, S), itemid_seq.dtype),
            jax.ShapeDtypeStruct((B, S), jnp.bool_),
            jax.ShapeDtypeStruct((B, S, S), jnp.float32),
            jax.ShapeDtypeStruct((B, S, S), jnp.int32),
        ],
        compiler_params=pltpu.CompilerParams(
            dimension_semantics=("parallel",),
        ),
    )(itemid_seq, pos_emb, memb)

    pos_emb_inp, labels, masked_labels, target_mapping, perm_mask_out = out
    return (pos_emb_inp, labels, masked_labels, target_mapping, perm_mask_out)
